# baseline (speedup 1.0000x reference)
import functools

import jax
import jax.numpy as jnp
from jax import lax
from jax.experimental import pallas as pl
from jax.experimental.pallas import tpu as pltpu
from jax.experimental.pallas import tpu_sc as plsc

NCORES = 2
NSUB = 16
NTILES = NCORES * NSUB
CHUNK = 128


def _sc_mesh():
    return plsc.VectorSubcoreMesh(
        core_axis_name="c", subcore_axis_name="s",
        num_cores=NCORES, num_subcores=NSUB)


_SC_PARAMS = pltpu.CompilerParams(use_tc_tiling_on_sc=False)


@functools.partial(jax.jit, static_argnames=("npad",))
def _deg_call(dst2d, npad):
    erows = dst2d.shape[0]
    rpt = erows // NTILES
    rps = npad // NSUB
    w = 16

    def body(dst_hbm, out_hbm, idx_v, ones_v, zbuf_v, acc_sh, sem):
        cid = lax.axis_index("c")
        sid = lax.axis_index("s")
        wid = cid * NSUB + sid

        def fill(i, _):
            ones_v[i, :] = jnp.ones((16,), jnp.float32)
            return 0
        lax.fori_loop(0, CHUNK, fill, 0)

        def zero(i, _):
            zbuf_v[i, :] = jnp.zeros((16,), jnp.float32)
            return 0
        lax.fori_loop(0, rps, zero, 0)
        pltpu.sync_copy(zbuf_v, acc_sh.at[pl.ds(sid * rps, rps)])
        pltpu.sync_copy(dst_hbm.at[pl.ds(wid * rpt, rpt)], idx_v)
        plsc.subcore_barrier()

        def edge(h, _):
            g = 4 * h
            cs = [pltpu.async_copy(ones_v, acc_sh.at[idx_v.at[g + j]], sem,
                                   add=True) for j in range(4)]
            for c in cs:
                c.wait()
            return 0
        lax.fori_loop(0, rpt // 4, edge, 0)
        plsc.subcore_barrier()
        pltpu.sync_copy(acc_sh.at[pl.ds(sid * rps, rps)],
                        out_hbm.at[cid, pl.ds(sid * rps, rps)])

    kern = pl.kernel(
        body,
        out_type=jax.ShapeDtypeStruct((NCORES, npad, w), jnp.float32),
        mesh=_sc_mesh(),
        compiler_params=_SC_PARAMS,
        scratch_types=[
            pltpu.VMEM((rpt, CHUNK), jnp.int32),
            pltpu.VMEM((CHUNK, w), jnp.float32),
            pltpu.VMEM((rps, w), jnp.float32),
            pltpu.VMEM_SHARED((npad, w), jnp.float32),
            pltpu.SemaphoreType.DMA,
        ])
    return kern(dst2d)


@functools.partial(jax.jit, static_argnames=("npad",))
def _segsum_call(table, src2d, dst2d, npad):
    erows = src2d.shape[0]
    rpt = erows // NTILES
    rps = npad // NSUB
    w = 32

    nbuf = 8
    dist = 4

    def body(table_hbm, src_hbm, dst_hbm, out_hbm,
             src_v, dst_v, rows, zbuf_v, table_sh, acc_sh, gsems, ssems):
        cid = lax.axis_index("c")
        sid = lax.axis_index("s")
        wid = cid * NSUB + sid

        def zero(i, _):
            zbuf_v[i, pl.ds(0, 16)] = jnp.zeros((16,), jnp.float32)
            zbuf_v[i, pl.ds(16, 16)] = jnp.zeros((16,), jnp.float32)
            return 0
        lax.fori_loop(0, rps, zero, 0)
        pltpu.sync_copy(zbuf_v, acc_sh.at[pl.ds(sid * rps, rps)])
        pltpu.sync_copy(table_hbm.at[pl.ds(sid * rps, rps)],
                        table_sh.at[pl.ds(sid * rps, rps)])
        pltpu.sync_copy(src_hbm.at[pl.ds(wid * rpt, rpt)], src_v)
        pltpu.sync_copy(dst_hbm.at[pl.ds(wid * rpt, rpt)], dst_v)
        plsc.subcore_barrier()

        def gather(g, b):
            pltpu.async_copy(table_sh.at[src_v.at[g]], rows[b], gsems[b])

        def gather_wait(g, b):
            pltpu.make_async_copy(table_sh.at[src_v.at[g]], rows[b],
                                  gsems[b]).wait()

        def scat(g, b):
            pltpu.async_copy(rows[b], acc_sh.at[dst_v.at[g]], ssems[b],
                             add=True)

        def scat_wait(g, b):
            pltpu.make_async_copy(rows[b], acc_sh.at[dst_v.at[g]],
                                  ssems[b]).wait()

        for j in range(dist):
            gather(j, j)

        def edge(h, _):
            for j in range(nbuf):
                g = nbuf * h + j
                b2 = (j + dist) % nbuf

                @pl.when(g >= dist)
                def _():
                    scat_wait(g - dist, b2)

                @pl.when(g + dist < rpt)
                def _():
                    gather(g + dist, b2)
                gather_wait(g, j)
                scat(g, j)
            return 0
        lax.fori_loop(0, rpt // nbuf, edge, 0)
        for j in range(dist):
            g = rpt - dist + j
            scat_wait(g, g % nbuf)
        plsc.subcore_barrier()
        pltpu.sync_copy(acc_sh.at[pl.ds(sid * rps, rps)],
                        out_hbm.at[cid, pl.ds(sid * rps, rps)])

    kern = pl.kernel(
        body,
        out_type=jax.ShapeDtypeStruct((NCORES, npad, w), jnp.float32),
        mesh=_sc_mesh(),
        compiler_params=_SC_PARAMS,
        scratch_types=[
            pltpu.VMEM((rpt, CHUNK), jnp.int32),
            pltpu.VMEM((rpt, CHUNK), jnp.int32),
            tuple(pltpu.VMEM((CHUNK, w), jnp.float32) for _ in range(nbuf)),
            pltpu.VMEM((rps, w), jnp.float32),
            pltpu.VMEM_SHARED((npad, w), jnp.float32),
            pltpu.VMEM_SHARED((npad, w), jnp.float32),
            tuple(pltpu.SemaphoreType.DMA for _ in range(nbuf)),
            tuple(pltpu.SemaphoreType.DMA for _ in range(nbuf)),
        ])
    return kern(table, src2d, dst2d)


def _tc0(x, w1, npad):
    n = x.shape[0]
    blk = npad // 4

    def body(x_ref, w1_ref, hw_ref):
        i = pl.program_id(0)
        hw = jnp.dot(x_ref[...], w1_ref[...], preferred_element_type=jnp.float32)
        rows = i * blk + lax.broadcasted_iota(jnp.int32, (blk, 1), 0)
        hw_ref[...] = jnp.where(rows < n, hw, 0.0)

    return pl.pallas_call(
        body,
        grid=(4,),
        in_specs=[
            pl.BlockSpec((blk, 128), lambda i: (i, 0)),
            pl.BlockSpec((128, 32), lambda i: (0, 0)),
        ],
        out_specs=pl.BlockSpec((blk, 32), lambda i: (i, 0)),
        out_shape=jax.ShapeDtypeStruct((npad, 32), jnp.float32),
    )(x, w1)


def _tc1s(degparts, hw):
    npad = hw.shape[0]
    blk = npad // 4

    def body(dp_ref, hw_ref, dinv_ref, hw2_ref):
        dp = dp_ref[0] + dp_ref[1]
        deg = dp[:, 0:1] + 1.0
        dinv = lax.rsqrt(deg)
        hw2_ref[...] = hw_ref[...] * dinv
        dinv_ref[...] = jnp.broadcast_to(dinv, (blk, 8))

    return pl.pallas_call(
        body,
        grid=(4,),
        in_specs=[
            pl.BlockSpec((NCORES, blk, 16), lambda i: (0, i, 0)),
            pl.BlockSpec((blk, 32), lambda i: (i, 0)),
        ],
        out_specs=[
            pl.BlockSpec((blk, 8), lambda i: (i, 0)),
            pl.BlockSpec((blk, 32), lambda i: (i, 0)),
        ],
        out_shape=[
            jax.ShapeDtypeStruct((npad, 8), jnp.float32),
            jax.ShapeDtypeStruct((npad, 32), jnp.float32),
        ],
    )(degparts, hw)


def _tc2(parts, hw2, dinv8, b1, w2, n_valid):
    npad = hw2.shape[0]
    blk = npad // 4

    def body(p_ref, hw2_ref, dinv_ref, b1_ref, w2_ref, out_ref):
        i = pl.program_id(0)
        dinv = dinv_ref[:, 0:1]
        s = p_ref[0] + p_ref[1] + hw2_ref[...]
        h1 = jnp.maximum(dinv * s + b1_ref[...], 0.0)
        hw2n = jnp.dot(h1, w2_ref[...], preferred_element_type=jnp.float32)
        hw2n = hw2n * dinv
        rows = i * blk + lax.broadcasted_iota(jnp.int32, (blk, 1), 0)
        out_ref[...] = jnp.where(rows < n_valid, hw2n, 0.0)

    return pl.pallas_call(
        body,
        grid=(4,),
        in_specs=[
            pl.BlockSpec((NCORES, blk, 32), lambda i: (0, i, 0)),
            pl.BlockSpec((blk, 32), lambda i: (i, 0)),
            pl.BlockSpec((blk, 8), lambda i: (i, 0)),
            pl.BlockSpec((1, 32), lambda i: (0, 0)),
            pl.BlockSpec((32, 32), lambda i: (0, 0)),
        ],
        out_specs=pl.BlockSpec((blk, 32), lambda i: (i, 0)),
        out_shape=jax.ShapeDtypeStruct((npad, 32), jnp.float32),
    )(parts, hw2, dinv8, b1, w2)


def _tc3(parts2, hw2p, dinv8, b2, n2v, w_iht, w_hht, b_ih, b_hh, w_lin, b_lin):
    t_dim, n, h = n2v.shape
    g = 2 * h
    z = w_lin.shape[1]
    blk = 2000
    grid = n // blk

    def body(p_ref, hw2_ref, dinv_ref, b2_ref, n2v_ref, wih_ref, whh_ref,
             bih_ref, bhh_ref, wlin_ref, blin_ref, out_ref):
        dinv = dinv_ref[:, 0:1]
        z2 = dinv * (p_ref[0] + p_ref[1] + hw2_ref[...]) + b2_ref[...]
        dn = (((1,), (1,)), ((), ()))
        xs_all = jnp.concatenate(
            [jnp.tanh(jnp.concatenate([z2, n2v_ref[t]], axis=1))
             for t in range(t_dim)], axis=0)
        gi_all = lax.dot_general(xs_all, wih_ref[...], dn,
                                 preferred_element_type=jnp.float32) + bih_ref[...]
        hstate = jnp.zeros((blk, g), jnp.float32)
        hs = []
        for t in range(t_dim):
            gi = gi_all[t * blk:(t + 1) * blk]
            gh = lax.dot_general(hstate, whh_ref[...], dn,
                                 preferred_element_type=jnp.float32) + bhh_ref[...]
            r = jax.nn.sigmoid(gi[:, 0:g] + gh[:, 0:g])
            zg = jax.nn.sigmoid(gi[:, g:2 * g] + gh[:, g:2 * g])
            cand = jnp.tanh(gi[:, 2 * g:3 * g] + r * gh[:, 2 * g:3 * g])
            hstate = (1.0 - zg) * cand + zg * hstate
            hs.append(hstate)
        res_all = jnp.dot(jnp.concatenate(hs, axis=0), wlin_ref[...],
                          preferred_element_type=jnp.float32) + blin_ref[...]
        for t in range(t_dim):
            out_ref[t] = res_all[t * blk:(t + 1) * blk]

    return pl.pallas_call(
        body,
        grid=(grid,),
        in_specs=[
            pl.BlockSpec((NCORES, blk, h), lambda i: (0, i, 0)),
            pl.BlockSpec((blk, h), lambda i: (i, 0)),
            pl.BlockSpec((blk, 8), lambda i: (i, 0)),
            pl.BlockSpec((1, h), lambda i: (0, 0)),
            pl.BlockSpec((t_dim, blk, h), lambda i: (0, i, 0)),
            pl.BlockSpec((3 * g, g), lambda i: (0, 0)),
            pl.BlockSpec((3 * g, g), lambda i: (0, 0)),
            pl.BlockSpec((1, 3 * g), lambda i: (0, 0)),
            pl.BlockSpec((1, 3 * g), lambda i: (0, 0)),
            pl.BlockSpec((g, z), lambda i: (0, 0)),
            pl.BlockSpec((1, z), lambda i: (0, 0)),
        ],
        out_specs=pl.BlockSpec((t_dim, blk, z), lambda i: (0, i, 0)),
        out_shape=jax.ShapeDtypeStruct((t_dim, n, z), jnp.float32),
    )(parts2, hw2p, dinv8, b2, n2v, w_iht, w_hht, b_ih, b_hh, w_lin, b_lin)


def kernel(x, edge_index, N2V_embeds, W1, b1, W2, b2,
           W_ih, W_hh, b_ih, b_hh, W_lin, b_lin):
    n, f = x.shape
    e = edge_index.shape[1]

    npad = ((n + 127) // 128) * 128
    epad = ((e + NTILES * CHUNK * 8 - 1) // (NTILES * CHUNK * 8)) * (NTILES * CHUNK * 8)

    padfill = (n + jnp.arange(epad - e, dtype=jnp.int32) % (npad - n))
    src = jnp.concatenate([edge_index[0], padfill]).reshape(epad // CHUNK, CHUNK)
    dst = jnp.concatenate([edge_index[1], padfill]).reshape(epad // CHUNK, CHUNK)

    degparts = _deg_call(dst, npad=npad)
    hw = _tc0(x, W1, npad)
    dinv8, hw2 = _tc1s(degparts, hw)
    parts1 = _segsum_call(hw2, src, dst, npad=npad)
    hw2p = _tc2(parts1, hw2, dinv8, b1.reshape(1, -1), W2, n)
    parts2 = _segsum_call(hw2p, src, dst, npad=npad)
    out = _tc3(parts2, hw2p, dinv8, b2.reshape(1, -1), N2V_embeds,
               W_ih, W_hh, b_ih.reshape(1, -1), b_hh.reshape(1, -1),
               W_lin, b_lin.reshape(1, -1))
    return out


# speedup vs baseline: 1.0237x; 1.0237x over previous
import functools

import jax
import jax.numpy as jnp
from jax import lax
from jax.experimental import pallas as pl
from jax.experimental.pallas import tpu as pltpu
from jax.experimental.pallas import tpu_sc as plsc

NCORES = 2
NSUB = 16
NTILES = NCORES * NSUB
CHUNK = 128


def _sc_mesh():
    return plsc.VectorSubcoreMesh(
        core_axis_name="c", subcore_axis_name="s",
        num_cores=NCORES, num_subcores=NSUB)


_SC_PARAMS = pltpu.CompilerParams(use_tc_tiling_on_sc=False)


@functools.partial(jax.jit, static_argnames=("npad",))
def _deg_call(e3d, npad):
    erows = e3d.shape[1]
    q, r = divmod(erows, NTILES)
    load_rows = q + (1 if r else 0)
    rps = npad // NSUB
    w = 16

    def body(e_hbm, out_hbm, idx_v, ones_v, zbuf_v, acc_sh, sem):
        cid = lax.axis_index("c")
        sid = lax.axis_index("s")
        wid = cid * NSUB + sid
        base = q * wid + jnp.maximum(0, wid - (NTILES - r))
        cnt = q + jnp.where(wid >= NTILES - r, 1, 0)

        def fill(i, _):
            ones_v[i, :] = jnp.ones((16,), jnp.float32)
            return 0
        lax.fori_loop(0, CHUNK, fill, 0)

        def zero(i, _):
            zbuf_v[i, :] = jnp.zeros((16,), jnp.float32)
            return 0
        lax.fori_loop(0, rps, zero, 0)
        pltpu.sync_copy(zbuf_v, acc_sh.at[pl.ds(sid * rps, rps)])
        pltpu.sync_copy(e_hbm.at[1, pl.ds(base, load_rows)], idx_v)
        plsc.subcore_barrier()

        def edge(h, _):
            g = 4 * h
            cs = [pltpu.async_copy(ones_v, acc_sh.at[idx_v.at[g + j]], sem,
                                   add=True) for j in range(4)]
            for c in cs:
                c.wait()
            return 0
        lax.fori_loop(0, q // 4, edge, 0)

        def rem(g, _):
            pltpu.sync_copy(ones_v, acc_sh.at[idx_v.at[g]], add=True)
            return 0
        lax.fori_loop(4 * (q // 4), cnt, rem, 0)
        plsc.subcore_barrier()
        pltpu.sync_copy(acc_sh.at[pl.ds(sid * rps, rps)],
                        out_hbm.at[cid, pl.ds(sid * rps, rps)])

    kern = pl.kernel(
        body,
        out_type=jax.ShapeDtypeStruct((NCORES, npad, w), jnp.float32),
        mesh=_sc_mesh(),
        compiler_params=_SC_PARAMS,
        scratch_types=[
            pltpu.VMEM((load_rows, CHUNK), jnp.int32),
            pltpu.VMEM((CHUNK, w), jnp.float32),
            pltpu.VMEM((rps, w), jnp.float32),
            pltpu.VMEM_SHARED((npad, w), jnp.float32),
            pltpu.SemaphoreType.DMA,
        ])
    return kern(e3d)


@functools.partial(jax.jit, static_argnames=("npad",))
def _segsum_call(table, src2d, dst2d, npad):
    erows = src2d.shape[0]
    rpt = erows // NTILES
    rps = npad // NSUB
    w = 32

    nbuf = 8
    dist = 4

    def body(table_hbm, src_hbm, dst_hbm, out_hbm,
             src_v, dst_v, rows, zbuf_v, table_sh, acc_sh, gsems, ssems):
        cid = lax.axis_index("c")
        sid = lax.axis_index("s")
        wid = cid * NSUB + sid

        def zero(i, _):
            zbuf_v[i, pl.ds(0, 16)] = jnp.zeros((16,), jnp.float32)
            zbuf_v[i, pl.ds(16, 16)] = jnp.zeros((16,), jnp.float32)
            return 0
        lax.fori_loop(0, rps, zero, 0)
        pltpu.sync_copy(zbuf_v, acc_sh.at[pl.ds(sid * rps, rps)])
        pltpu.sync_copy(table_hbm.at[pl.ds(sid * rps, rps)],
                        table_sh.at[pl.ds(sid * rps, rps)])
        pltpu.sync_copy(src_hbm.at[pl.ds(wid * rpt, rpt)], src_v)
        pltpu.sync_copy(dst_hbm.at[pl.ds(wid * rpt, rpt)], dst_v)
        plsc.subcore_barrier()

        def gather(g, b):
            pltpu.async_copy(table_sh.at[src_v.at[g]], rows[b], gsems[b])

        def gather_wait(g, b):
            pltpu.make_async_copy(table_sh.at[src_v.at[g]], rows[b],
                                  gsems[b]).wait()

        def scat(g, b):
            pltpu.async_copy(rows[b], acc_sh.at[dst_v.at[g]], ssems[b],
                             add=True)

        def scat_wait(g, b):
            pltpu.make_async_copy(rows[b], acc_sh.at[dst_v.at[g]],
                                  ssems[b]).wait()

        for j in range(dist):
            gather(j, j)

        def edge(h, _):
            for j in range(nbuf):
                g = nbuf * h + j
                b2 = (j + dist) % nbuf

                @pl.when(g >= dist)
                def _():
                    scat_wait(g - dist, b2)

                @pl.when(g + dist < rpt)
                def _():
                    gather(g + dist, b2)
                gather_wait(g, j)
                scat(g, j)
            return 0
        lax.fori_loop(0, rpt // nbuf, edge, 0)
        for j in range(dist):
            g = rpt - dist + j
            scat_wait(g, g % nbuf)
        plsc.subcore_barrier()
        pltpu.sync_copy(acc_sh.at[pl.ds(sid * rps, rps)],
                        out_hbm.at[cid, pl.ds(sid * rps, rps)])

    kern = pl.kernel(
        body,
        out_type=jax.ShapeDtypeStruct((NCORES, npad, w), jnp.float32),
        mesh=_sc_mesh(),
        compiler_params=_SC_PARAMS,
        scratch_types=[
            pltpu.VMEM((rpt, CHUNK), jnp.int32),
            pltpu.VMEM((rpt, CHUNK), jnp.int32),
            tuple(pltpu.VMEM((CHUNK, w), jnp.float32) for _ in range(nbuf)),
            pltpu.VMEM((rps, w), jnp.float32),
            pltpu.VMEM_SHARED((npad, w), jnp.float32),
            pltpu.VMEM_SHARED((npad, w), jnp.float32),
            tuple(pltpu.SemaphoreType.DMA for _ in range(nbuf)),
            tuple(pltpu.SemaphoreType.DMA for _ in range(nbuf)),
        ])
    return kern(table, src2d, dst2d)


def _tc0(x, w1, npad):
    n = x.shape[0]
    blk = npad // 4

    def body(x_ref, w1_ref, hw_ref):
        i = pl.program_id(0)
        hw = jnp.dot(x_ref[...], w1_ref[...], preferred_element_type=jnp.float32)
        rows = i * blk + lax.broadcasted_iota(jnp.int32, (blk, 1), 0)
        hw_ref[...] = jnp.where(rows < n, hw, 0.0)

    return pl.pallas_call(
        body,
        grid=(4,),
        in_specs=[
            pl.BlockSpec((blk, 128), lambda i: (i, 0)),
            pl.BlockSpec((128, 32), lambda i: (0, 0)),
        ],
        out_specs=pl.BlockSpec((blk, 32), lambda i: (i, 0)),
        out_shape=jax.ShapeDtypeStruct((npad, 32), jnp.float32),
    )(x, w1)


def _tc1s(degparts, hw):
    npad = hw.shape[0]
    blk = npad // 4

    def body(dp_ref, hw_ref, dinv_ref, hw2_ref):
        dp = dp_ref[0] + dp_ref[1]
        deg = dp[:, 0:1] + 1.0
        dinv = lax.rsqrt(deg)
        hw2_ref[...] = hw_ref[...] * dinv
        dinv_ref[...] = jnp.broadcast_to(dinv, (blk, 8))

    return pl.pallas_call(
        body,
        grid=(4,),
        in_specs=[
            pl.BlockSpec((NCORES, blk, 16), lambda i: (0, i, 0)),
            pl.BlockSpec((blk, 32), lambda i: (i, 0)),
        ],
        out_specs=[
            pl.BlockSpec((blk, 8), lambda i: (i, 0)),
            pl.BlockSpec((blk, 32), lambda i: (i, 0)),
        ],
        out_shape=[
            jax.ShapeDtypeStruct((npad, 8), jnp.float32),
            jax.ShapeDtypeStruct((npad, 32), jnp.float32),
        ],
    )(degparts, hw)


def _tc2(parts, hw2, dinv8, b1, w2, n_valid):
    npad = hw2.shape[0]
    blk = npad // 4

    def body(p_ref, hw2_ref, dinv_ref, b1_ref, w2_ref, out_ref):
        i = pl.program_id(0)
        dinv = dinv_ref[:, 0:1]
        s = p_ref[0] + p_ref[1] + hw2_ref[...]
        h1 = jnp.maximum(dinv * s + b1_ref[...], 0.0)
        hw2n = jnp.dot(h1, w2_ref[...], preferred_element_type=jnp.float32)
        hw2n = hw2n * dinv
        rows = i * blk + lax.broadcasted_iota(jnp.int32, (blk, 1), 0)
        out_ref[...] = jnp.where(rows < n_valid, hw2n, 0.0)

    return pl.pallas_call(
        body,
        grid=(4,),
        in_specs=[
            pl.BlockSpec((NCORES, blk, 32), lambda i: (0, i, 0)),
            pl.BlockSpec((blk, 32), lambda i: (i, 0)),
            pl.BlockSpec((blk, 8), lambda i: (i, 0)),
            pl.BlockSpec((1, 32), lambda i: (0, 0)),
            pl.BlockSpec((32, 32), lambda i: (0, 0)),
        ],
        out_specs=pl.BlockSpec((blk, 32), lambda i: (i, 0)),
        out_shape=jax.ShapeDtypeStruct((npad, 32), jnp.float32),
    )(parts, hw2, dinv8, b1, w2)


def _tc3(parts2, hw2p, dinv8, b2, n2v, w_iht, w_hht, b_ih, b_hh, w_lin, b_lin):
    t_dim, n, h = n2v.shape
    g = 2 * h
    z = w_lin.shape[1]
    blk = 2000
    grid = n // blk

    def body(p_ref, hw2_ref, dinv_ref, b2_ref, n2v_ref, wih_ref, whh_ref,
             bih_ref, bhh_ref, wlin_ref, blin_ref, out_ref):
        dinv = dinv_ref[:, 0:1]
        z2 = dinv * (p_ref[0] + p_ref[1] + hw2_ref[...]) + b2_ref[...]
        dn = (((1,), (1,)), ((), ()))
        xs_all = jnp.concatenate(
            [jnp.tanh(jnp.concatenate([z2, n2v_ref[t]], axis=1))
             for t in range(t_dim)], axis=0)
        gi_all = lax.dot_general(xs_all, wih_ref[...], dn,
                                 preferred_element_type=jnp.float32) + bih_ref[...]
        hstate = jnp.zeros((blk, g), jnp.float32)
        hs = []
        for t in range(t_dim):
            gi = gi_all[t * blk:(t + 1) * blk]
            gh = lax.dot_general(hstate, whh_ref[...], dn,
                                 preferred_element_type=jnp.float32) + bhh_ref[...]
            r = jax.nn.sigmoid(gi[:, 0:g] + gh[:, 0:g])
            zg = jax.nn.sigmoid(gi[:, g:2 * g] + gh[:, g:2 * g])
            cand = jnp.tanh(gi[:, 2 * g:3 * g] + r * gh[:, 2 * g:3 * g])
            hstate = (1.0 - zg) * cand + zg * hstate
            hs.append(hstate)
        res_all = jnp.dot(jnp.concatenate(hs, axis=0), wlin_ref[...],
                          preferred_element_type=jnp.float32) + blin_ref[...]
        for t in range(t_dim):
            out_ref[t] = res_all[t * blk:(t + 1) * blk]

    return pl.pallas_call(
        body,
        grid=(grid,),
        in_specs=[
            pl.BlockSpec((NCORES, blk, h), lambda i: (0, i, 0)),
            pl.BlockSpec((blk, h), lambda i: (i, 0)),
            pl.BlockSpec((blk, 8), lambda i: (i, 0)),
            pl.BlockSpec((1, h), lambda i: (0, 0)),
            pl.BlockSpec((t_dim, blk, h), lambda i: (0, i, 0)),
            pl.BlockSpec((3 * g, g), lambda i: (0, 0)),
            pl.BlockSpec((3 * g, g), lambda i: (0, 0)),
            pl.BlockSpec((1, 3 * g), lambda i: (0, 0)),
            pl.BlockSpec((1, 3 * g), lambda i: (0, 0)),
            pl.BlockSpec((g, z), lambda i: (0, 0)),
            pl.BlockSpec((1, z), lambda i: (0, 0)),
        ],
        out_specs=pl.BlockSpec((t_dim, blk, z), lambda i: (0, i, 0)),
        out_shape=jax.ShapeDtypeStruct((t_dim, n, z), jnp.float32),
    )(parts2, hw2p, dinv8, b2, n2v, w_iht, w_hht, b_ih, b_hh, w_lin, b_lin)


def kernel(x, edge_index, N2V_embeds, W1, b1, W2, b2,
           W_ih, W_hh, b_ih, b_hh, W_lin, b_lin):
    n, f = x.shape
    e = edge_index.shape[1]

    npad = ((n + 127) // 128) * 128
    epad = ((e + NTILES * CHUNK * 8 - 1) // (NTILES * CHUNK * 8)) * (NTILES * CHUNK * 8)

    degparts = _deg_call(edge_index.reshape(2, e // CHUNK, CHUNK), npad=npad)

    padfill = (n + jnp.arange(epad - e, dtype=jnp.int32) % (npad - n))
    src = jnp.concatenate([edge_index[0], padfill]).reshape(epad // CHUNK, CHUNK)
    dst = jnp.concatenate([edge_index[1], padfill]).reshape(epad // CHUNK, CHUNK)

    hw = _tc0(x, W1, npad)
    dinv8, hw2 = _tc1s(degparts, hw)
    parts1 = _segsum_call(hw2, src, dst, npad=npad)
    hw2p = _tc2(parts1, hw2, dinv8, b1.reshape(1, -1), W2, n)
    parts2 = _segsum_call(hw2p, src, dst, npad=npad)
    out = _tc3(parts2, hw2p, dinv8, b2.reshape(1, -1), N2V_embeds,
               W_ih, W_hh, b_ih.reshape(1, -1), b_hh.reshape(1, -1),
               W_lin, b_lin.reshape(1, -1))
    return out


# speedup vs baseline: 1.0761x; 1.0511x over previous
import functools

import jax
import jax.numpy as jnp
from jax import lax
from jax.experimental import pallas as pl
from jax.experimental.pallas import tpu as pltpu
from jax.experimental.pallas import tpu_sc as plsc

NCORES = 2
NSUB = 16
NTILES = NCORES * NSUB
CHUNK = 128


def _sc_mesh():
    return plsc.VectorSubcoreMesh(
        core_axis_name="c", subcore_axis_name="s",
        num_cores=NCORES, num_subcores=NSUB)


_SC_PARAMS = pltpu.CompilerParams(use_tc_tiling_on_sc=False)


@functools.partial(jax.jit, static_argnames=("npad",))
def _deg_call(e3d, npad):
    erows = e3d.shape[1]
    q, r = divmod(erows, NTILES)
    load_rows = q + (1 if r else 0)
    rps = npad // NSUB
    w = 16

    def body(e_hbm, out_hbm, idx_v, ones_v, zbuf_v, acc_sh, sem):
        cid = lax.axis_index("c")
        sid = lax.axis_index("s")
        wid = cid * NSUB + sid
        base = q * wid + jnp.maximum(0, wid - (NTILES - r))
        cnt = q + jnp.where(wid >= NTILES - r, 1, 0)

        def fill(i, _):
            ones_v[i, :] = jnp.ones((16,), jnp.float32)
            return 0
        lax.fori_loop(0, CHUNK, fill, 0)

        def zero(i, _):
            zbuf_v[i, :] = jnp.zeros((16,), jnp.float32)
            return 0
        lax.fori_loop(0, rps, zero, 0)
        pltpu.sync_copy(zbuf_v, acc_sh.at[pl.ds(sid * rps, rps)])
        pltpu.sync_copy(e_hbm.at[1, pl.ds(base, load_rows)], idx_v)
        plsc.subcore_barrier()

        def edge(h, _):
            g = 4 * h
            cs = [pltpu.async_copy(ones_v, acc_sh.at[idx_v.at[g + j]], sem,
                                   add=True) for j in range(4)]
            for c in cs:
                c.wait()
            return 0
        lax.fori_loop(0, q // 4, edge, 0)

        def rem(g, _):
            pltpu.sync_copy(ones_v, acc_sh.at[idx_v.at[g]], add=True)
            return 0
        lax.fori_loop(4 * (q // 4), cnt, rem, 0)
        plsc.subcore_barrier()
        pltpu.sync_copy(acc_sh.at[pl.ds(sid * rps, rps)],
                        out_hbm.at[cid, pl.ds(sid * rps, rps)])

    kern = pl.kernel(
        body,
        out_type=jax.ShapeDtypeStruct((NCORES, npad, w), jnp.float32),
        mesh=_sc_mesh(),
        compiler_params=_SC_PARAMS,
        scratch_types=[
            pltpu.VMEM((load_rows, CHUNK), jnp.int32),
            pltpu.VMEM((CHUNK, w), jnp.float32),
            pltpu.VMEM((rps, w), jnp.float32),
            pltpu.VMEM_SHARED((npad, w), jnp.float32),
            pltpu.SemaphoreType.DMA,
        ])
    return kern(e3d)


@functools.partial(jax.jit, static_argnames=("npad",))
def _segsum_call(table, src2d, dst2d, npad):
    erows = src2d.shape[0]
    rpt = erows // NTILES
    rps = npad // NSUB
    w = 32

    nbuf = 8
    dist = 4

    def body(table_hbm, src_hbm, dst_hbm, out_hbm,
             src_v, dst_v, rows, zbuf_v, table_sh, acc_sh, gsems, ssems):
        cid = lax.axis_index("c")
        sid = lax.axis_index("s")
        wid = cid * NSUB + sid

        tcp = pltpu.async_copy(table_hbm.at[pl.ds(sid * rps, rps)],
                               table_sh.at[pl.ds(sid * rps, rps)], gsems[0])
        scp = pltpu.async_copy(src_hbm.at[pl.ds(wid * rpt, rpt)], src_v,
                               gsems[1])
        dcp = pltpu.async_copy(dst_hbm.at[pl.ds(wid * rpt, rpt)], dst_v,
                               gsems[2])

        def zero(i, _):
            zbuf_v[i, pl.ds(0, 16)] = jnp.zeros((16,), jnp.float32)
            zbuf_v[i, pl.ds(16, 16)] = jnp.zeros((16,), jnp.float32)
            return 0
        lax.fori_loop(0, rps, zero, 0)
        pltpu.sync_copy(zbuf_v, acc_sh.at[pl.ds(sid * rps, rps)])
        tcp.wait()
        scp.wait()
        dcp.wait()
        plsc.subcore_barrier()

        def gather(g, b):
            pltpu.async_copy(table_sh.at[src_v.at[g]], rows[b], gsems[b])

        def gather_wait(g, b):
            pltpu.make_async_copy(table_sh.at[src_v.at[g]], rows[b],
                                  gsems[b]).wait()

        def scat(g, b):
            pltpu.async_copy(rows[b], acc_sh.at[dst_v.at[g]], ssems[b],
                             add=True)

        def scat_wait(g, b):
            pltpu.make_async_copy(rows[b], acc_sh.at[dst_v.at[g]],
                                  ssems[b]).wait()

        for j in range(dist):
            gather(j, j)

        def edge(h, _):
            for j in range(nbuf):
                g = nbuf * h + j
                b2 = (j + dist) % nbuf

                @pl.when(g >= dist)
                def _():
                    scat_wait(g - dist, b2)

                @pl.when(g + dist < rpt)
                def _():
                    gather(g + dist, b2)
                gather_wait(g, j)
                scat(g, j)
            return 0
        lax.fori_loop(0, rpt // nbuf, edge, 0)
        for j in range(dist):
            g = rpt - dist + j
            scat_wait(g, g % nbuf)
        plsc.subcore_barrier()
        pltpu.sync_copy(acc_sh.at[pl.ds(sid * rps, rps)],
                        out_hbm.at[cid, pl.ds(sid * rps, rps)])

    kern = pl.kernel(
        body,
        out_type=jax.ShapeDtypeStruct((NCORES, npad, w), jnp.float32),
        mesh=_sc_mesh(),
        compiler_params=_SC_PARAMS,
        scratch_types=[
            pltpu.VMEM((rpt, CHUNK), jnp.int32),
            pltpu.VMEM((rpt, CHUNK), jnp.int32),
            tuple(pltpu.VMEM((CHUNK, w), jnp.float32) for _ in range(nbuf)),
            pltpu.VMEM((rps, w), jnp.float32),
            pltpu.VMEM_SHARED((npad, w), jnp.float32),
            pltpu.VMEM_SHARED((npad, w), jnp.float32),
            tuple(pltpu.SemaphoreType.DMA for _ in range(nbuf)),
            tuple(pltpu.SemaphoreType.DMA for _ in range(nbuf)),
        ])
    return kern(table, src2d, dst2d)


def _tc0(x, w1, npad):
    n = x.shape[0]
    blk = npad // 4

    def body(x_ref, w1_ref, hw_ref):
        i = pl.program_id(0)
        hw = jnp.dot(x_ref[...], w1_ref[...], preferred_element_type=jnp.float32)
        rows = i * blk + lax.broadcasted_iota(jnp.int32, (blk, 1), 0)
        hw_ref[...] = jnp.where(rows < n, hw, 0.0)

    return pl.pallas_call(
        body,
        grid=(4,),
        in_specs=[
            pl.BlockSpec((blk, 128), lambda i: (i, 0)),
            pl.BlockSpec((128, 32), lambda i: (0, 0)),
        ],
        out_specs=pl.BlockSpec((blk, 32), lambda i: (i, 0)),
        out_shape=jax.ShapeDtypeStruct((npad, 32), jnp.float32),
    )(x, w1)


def _tc1s(degparts, hw):
    npad = hw.shape[0]
    blk = npad // 4

    def body(dp_ref, hw_ref, dinv_ref, hw2_ref):
        dp = dp_ref[0] + dp_ref[1]
        deg = dp[:, 0:1] + 1.0
        dinv = lax.rsqrt(deg)
        hw2_ref[...] = hw_ref[...] * dinv
        dinv_ref[...] = jnp.broadcast_to(dinv, (blk, 8))

    return pl.pallas_call(
        body,
        grid=(4,),
        in_specs=[
            pl.BlockSpec((NCORES, blk, 16), lambda i: (0, i, 0)),
            pl.BlockSpec((blk, 32), lambda i: (i, 0)),
        ],
        out_specs=[
            pl.BlockSpec((blk, 8), lambda i: (i, 0)),
            pl.BlockSpec((blk, 32), lambda i: (i, 0)),
        ],
        out_shape=[
            jax.ShapeDtypeStruct((npad, 8), jnp.float32),
            jax.ShapeDtypeStruct((npad, 32), jnp.float32),
        ],
    )(degparts, hw)


def _tc2(parts, hw2, dinv8, b1, w2, n_valid):
    npad = hw2.shape[0]
    blk = npad // 4

    def body(p_ref, hw2_ref, dinv_ref, b1_ref, w2_ref, out_ref):
        i = pl.program_id(0)
        dinv = dinv_ref[:, 0:1]
        s = p_ref[0] + p_ref[1] + hw2_ref[...]
        h1 = jnp.maximum(dinv * s + b1_ref[...], 0.0)
        hw2n = jnp.dot(h1, w2_ref[...], preferred_element_type=jnp.float32)
        hw2n = hw2n * dinv
        rows = i * blk + lax.broadcasted_iota(jnp.int32, (blk, 1), 0)
        out_ref[...] = jnp.where(rows < n_valid, hw2n, 0.0)

    return pl.pallas_call(
        body,
        grid=(4,),
        in_specs=[
            pl.BlockSpec((NCORES, blk, 32), lambda i: (0, i, 0)),
            pl.BlockSpec((blk, 32), lambda i: (i, 0)),
            pl.BlockSpec((blk, 8), lambda i: (i, 0)),
            pl.BlockSpec((1, 32), lambda i: (0, 0)),
            pl.BlockSpec((32, 32), lambda i: (0, 0)),
        ],
        out_specs=pl.BlockSpec((blk, 32), lambda i: (i, 0)),
        out_shape=jax.ShapeDtypeStruct((npad, 32), jnp.float32),
    )(parts, hw2, dinv8, b1, w2)


def _tc3(parts2, hw2p, dinv8, b2, n2v, w_iht, w_hht, b_ih, b_hh, w_lin, b_lin):
    t_dim, n, h = n2v.shape
    g = 2 * h
    z = w_lin.shape[1]
    blk = 2000
    grid = n // blk

    def body(p_ref, hw2_ref, dinv_ref, b2_ref, n2v_ref, wih_ref, whh_ref,
             bih_ref, bhh_ref, wlin_ref, blin_ref, out_ref):
        dinv = dinv_ref[:, 0:1]
        z2 = dinv * (p_ref[0] + p_ref[1] + hw2_ref[...]) + b2_ref[...]
        dn = (((1,), (1,)), ((), ()))
        xs_all = jnp.concatenate(
            [jnp.tanh(jnp.concatenate([z2, n2v_ref[t]], axis=1))
             for t in range(t_dim)], axis=0)
        gi_all = lax.dot_general(xs_all, wih_ref[...], dn,
                                 preferred_element_type=jnp.float32) + bih_ref[...]
        hstate = jnp.zeros((blk, g), jnp.float32)
        hs = []
        for t in range(t_dim):
            gi = gi_all[t * blk:(t + 1) * blk]
            gh = lax.dot_general(hstate, whh_ref[...], dn,
                                 preferred_element_type=jnp.float32) + bhh_ref[...]
            r = jax.nn.sigmoid(gi[:, 0:g] + gh[:, 0:g])
            zg = jax.nn.sigmoid(gi[:, g:2 * g] + gh[:, g:2 * g])
            cand = jnp.tanh(gi[:, 2 * g:3 * g] + r * gh[:, 2 * g:3 * g])
            hstate = (1.0 - zg) * cand + zg * hstate
            hs.append(hstate)
        res_all = jnp.dot(jnp.concatenate(hs, axis=0), wlin_ref[...],
                          preferred_element_type=jnp.float32) + blin_ref[...]
        for t in range(t_dim):
            out_ref[t] = res_all[t * blk:(t + 1) * blk]

    return pl.pallas_call(
        body,
        grid=(grid,),
        in_specs=[
            pl.BlockSpec((NCORES, blk, h), lambda i: (0, i, 0)),
            pl.BlockSpec((blk, h), lambda i: (i, 0)),
            pl.BlockSpec((blk, 8), lambda i: (i, 0)),
            pl.BlockSpec((1, h), lambda i: (0, 0)),
            pl.BlockSpec((t_dim, blk, h), lambda i: (0, i, 0)),
            pl.BlockSpec((3 * g, g), lambda i: (0, 0)),
            pl.BlockSpec((3 * g, g), lambda i: (0, 0)),
            pl.BlockSpec((1, 3 * g), lambda i: (0, 0)),
            pl.BlockSpec((1, 3 * g), lambda i: (0, 0)),
            pl.BlockSpec((g, z), lambda i: (0, 0)),
            pl.BlockSpec((1, z), lambda i: (0, 0)),
        ],
        out_specs=pl.BlockSpec((t_dim, blk, z), lambda i: (0, i, 0)),
        out_shape=jax.ShapeDtypeStruct((t_dim, n, z), jnp.float32),
    )(parts2, hw2p, dinv8, b2, n2v, w_iht, w_hht, b_ih, b_hh, w_lin, b_lin)


def kernel(x, edge_index, N2V_embeds, W1, b1, W2, b2,
           W_ih, W_hh, b_ih, b_hh, W_lin, b_lin):
    n, f = x.shape
    e = edge_index.shape[1]

    npad = ((n + 127) // 128) * 128
    epad = ((e + NTILES * CHUNK * 8 - 1) // (NTILES * CHUNK * 8)) * (NTILES * CHUNK * 8)

    degparts = _deg_call(edge_index.reshape(2, e // CHUNK, CHUNK), npad=npad)

    padfill = (n + jnp.arange(epad - e, dtype=jnp.int32) % (npad - n))
    src = jnp.concatenate([edge_index[0], padfill]).reshape(epad // CHUNK, CHUNK)
    dst = jnp.concatenate([edge_index[1], padfill]).reshape(epad // CHUNK, CHUNK)

    hw = _tc0(x, W1, npad)
    dinv8, hw2 = _tc1s(degparts, hw)
    parts1 = _segsum_call(hw2, src, dst, npad=npad)
    hw2p = _tc2(parts1, hw2, dinv8, b1.reshape(1, -1), W2, n)
    parts2 = _segsum_call(hw2p, src, dst, npad=npad)
    out = _tc3(parts2, hw2p, dinv8, b2.reshape(1, -1), N2V_embeds,
               W_ih, W_hh, b_ih.reshape(1, -1), b_hh.reshape(1, -1),
               W_lin, b_lin.reshape(1, -1))
    return out
